# baseline (speedup 1.0000x reference)
import jax
import jax.numpy as jnp
from jax import lax
from jax.experimental import pallas as pl
from jax.experimental.pallas import tpu as pltpu
from jax.experimental.pallas import tpu_sc as plsc

_VOCAB = 1000
_DIM = 64
_BATCH = 4096
_SEQ = 200

_NC = 2
_NS = 16
_NW = _NC * _NS

_BW = _BATCH // _NW
_SBLK = 8
_NSB = _SEQ // _SBLK
_L = 16
_NG = _BW // _L


def _lookup_body(idx_hbm, tab_hbm, out_hbm, tab_v, idx_v, buf_v, sem):
    wid = lax.axis_index("s") * _NC + lax.axis_index("c")
    bw = wid * _BW

    pltpu.sync_copy(tab_hbm, tab_v)

    def seq_block(sb, carry):
        s0 = sb * _SBLK
        pltpu.sync_copy(idx_hbm.at[pl.ds(s0, _SBLK), pl.ds(bw, _BW)], idx_v)

        def seq_one(si, carry2):
            def group(g, carry3):
                idx16 = idx_v[si, pl.ds(g * _L, _L)]
                base = idx16 * _DIM
                for d in range(_DIM):
                    vals = plsc.load_gather(tab_v, [base + d])
                    buf_v[d, pl.ds(g * _L, _L)] = vals
                return carry3

            lax.fori_loop(0, _NG, group, 0, unroll=False)
            pltpu.sync_copy(buf_v, out_hbm.at[s0 + si, :, pl.ds(bw, _BW)])
            return carry2

        lax.fori_loop(0, _SBLK, seq_one, 0, unroll=False)
        return carry

    lax.fori_loop(0, _NSB, seq_block, 0, unroll=False)


@jax.jit
def _embedding_lookup(idx_t, tab_flat):
    mesh = plsc.VectorSubcoreMesh(
        core_axis_name="c", subcore_axis_name="s",
        num_cores=_NC, num_subcores=_NS,
    )
    return pl.kernel(
        _lookup_body,
        out_type=jax.ShapeDtypeStruct((_SEQ, _DIM, _BATCH), jnp.float32),
        mesh=mesh,
        scratch_types=[
            pltpu.VMEM((_VOCAB * _DIM,), jnp.float32),
            pltpu.VMEM((_SBLK, _BW), jnp.int32),
            pltpu.VMEM((_DIM, _BW), jnp.float32),
            pltpu.SemaphoreType.DMA,
        ],
        compiler_params=pltpu.CompilerParams(
            use_tc_tiling_on_sc=False, needs_layout_passes=False),
    )(idx_t, tab_flat)


def kernel(word_sequences, embedding_weight):
    idx_t = word_sequences.T
    tab_flat = embedding_weight.reshape(_VOCAB * _DIM)
    out = _embedding_lookup(idx_t, tab_flat)
    return jnp.transpose(out, (2, 0, 1))


# speedup vs baseline: 1.6772x; 1.6772x over previous
import jax
import jax.numpy as jnp
from jax import lax
from jax.experimental import pallas as pl
from jax.experimental.pallas import tpu as pltpu
from jax.experimental.pallas import tpu_sc as plsc

_VOCAB = 1000
_DIM = 64
_BATCH = 4096
_SEQ = 200

_NC = 2
_NS = 16
_NW = _NC * _NS

_BW = _BATCH // _NW
_SBLK = 8
_NSB = _SEQ // _SBLK
_L = 16
_NG = _BW // _L
_DB = 8


def _lookup_body(idx_hbm, tab_hbm, out_hbm, tab_v, idx_v, buf_v,
                 sem_idx, sem_out0, sem_out1):
    wid = lax.axis_index("s") * _NC + lax.axis_index("c")
    bw = wid * _BW

    pltpu.sync_copy(tab_hbm, tab_v)
    pltpu.async_copy(idx_hbm.at[pl.ds(0, _SBLK), pl.ds(bw, _BW)],
                     idx_v.at[0], sem_idx)

    def sb_body(sb, carry):
        blk = sb % 2
        pltpu.make_async_copy(
            idx_hbm.at[pl.ds(0, _SBLK), pl.ds(bw, _BW)],
            idx_v.at[blk], sem_idx).wait()

        @pl.when(sb < _NSB - 1)
        def _prefetch():
            pltpu.async_copy(
                idx_hbm.at[pl.ds((sb + 1) * _SBLK, _SBLK), pl.ds(bw, _BW)],
                idx_v.at[(sb + 1) % 2], sem_idx)

        for si in range(_SBLK):
            s = sb * _SBLK + si
            p = si % 2
            sem_out = sem_out0 if p == 0 else sem_out1

            @pl.when(s >= 2)
            def _drain():
                pltpu.make_async_copy(
                    buf_v.at[p], out_hbm.at[s - 2, :, pl.ds(bw, _BW)],
                    sem_out).wait()

            @plsc.parallel_loop(0, _NG)
            def _group(g):
                idx16 = idx_v[blk, si, pl.ds(g * _L, _L)]
                base = idx16 * _DIM
                for d0 in range(0, _DIM, _DB):
                    vals = [plsc.load_gather(tab_v, [base + (d0 + j)])
                            for j in range(_DB)]
                    for j in range(_DB):
                        buf_v[p, d0 + j, pl.ds(g * _L, _L)] = vals[j]

            pltpu.async_copy(buf_v.at[p], out_hbm.at[s, :, pl.ds(bw, _BW)],
                             sem_out)
        return carry

    lax.fori_loop(0, _NSB, sb_body, 0, unroll=False)

    pltpu.make_async_copy(buf_v.at[0], out_hbm.at[_SEQ - 2, :, pl.ds(bw, _BW)],
                          sem_out0).wait()
    pltpu.make_async_copy(buf_v.at[1], out_hbm.at[_SEQ - 1, :, pl.ds(bw, _BW)],
                          sem_out1).wait()


@jax.jit
def _embedding_lookup(idx_t, tab_flat):
    mesh = plsc.VectorSubcoreMesh(
        core_axis_name="c", subcore_axis_name="s",
        num_cores=_NC, num_subcores=_NS,
    )
    return pl.kernel(
        _lookup_body,
        out_type=jax.ShapeDtypeStruct((_SEQ, _DIM, _BATCH), jnp.float32),
        mesh=mesh,
        scratch_types=[
            pltpu.VMEM((_VOCAB * _DIM,), jnp.float32),
            pltpu.VMEM((2, _SBLK, _BW), jnp.int32),
            pltpu.VMEM((2, _DIM, _BW), jnp.float32),
            pltpu.SemaphoreType.DMA,
            pltpu.SemaphoreType.DMA,
            pltpu.SemaphoreType.DMA,
        ],
        compiler_params=pltpu.CompilerParams(
            use_tc_tiling_on_sc=False, needs_layout_passes=False),
    )(idx_t, tab_flat)


def kernel(word_sequences, embedding_weight):
    idx_t = word_sequences.T
    tab_flat = embedding_weight.reshape(_VOCAB * _DIM)
    out = _embedding_lookup(idx_t, tab_flat)
    return jnp.transpose(out, (2, 0, 1))
